# baseline (device time: 36616 ns/iter reference)
import functools

import jax
import jax.numpy as jnp
from jax import lax
from jax.experimental import pallas as pl
from jax.experimental.pallas import tpu as pltpu

N_DEV = 16
B, Sq, Hq, Dh = 2, 128, 4, 64
SKV_LOC = 128
BH = B * Hq
ROWS = BH * Sq
D_MODEL = 512
NEG = -1e9
R_HOPS = 4
L_HOPS = 3


def kernel(x, Wq, K_ext, V_ext, Wo):
    def body(x_ref, wq_ref, k_ref, v_ref, wo_ref, out_ref,
             cRa, cRm, cLa, cLm,
             aRs, aRr, mRs, mRr, aLs, aLr, mLs, mLr,
             fas, far, fms, fmr, gas, gar, gms, gmr,
             racc, rml, sacc, sml):
        my = lax.axis_index("i")
        is_even = (my % 2) == 0
        partner = jnp.where(is_even, my + 1, my - 1)

        def kof(pos):
            return jnp.where(pos % 4 == 0, pos // 4, 4 + (14 - pos) // 4)

        def cyc(kk):
            return jnp.where(kk <= 3, 4 * kk, 30 - 4 * kk)

        k_idx = kof(my)
        right_e = cyc((k_idx + 1) % 8)
        left_e = cyc((k_idx + 7) % 8)
        lt = cyc((k_idx + 4) % 8) + 1
        src_e = cyc((kof(partner) + 4) % 8)

        bsem = pltpu.get_barrier_semaphore()

        fa = pltpu.make_async_remote_copy(
            src_ref=sacc, dst_ref=racc, send_sem=fas, recv_sem=far,
            device_id=(partner,), device_id_type=pl.DeviceIdType.MESH)
        fm = pltpu.make_async_remote_copy(
            src_ref=sml, dst_ref=rml, send_sem=fms, recv_sem=fmr,
            device_id=(partner,), device_id_type=pl.DeviceIdType.MESH)
        ga = pltpu.make_async_remote_copy(
            src_ref=cRa.at[0], dst_ref=cRa.at[R_HOPS], send_sem=gas,
            recv_sem=gar, device_id=(lt,),
            device_id_type=pl.DeviceIdType.MESH)
        gm = pltpu.make_async_remote_copy(
            src_ref=cRm.at[0], dst_ref=cRm.at[R_HOPS], send_sem=gms,
            recv_sem=gmr, device_id=(lt,),
            device_id_type=pl.DeviceIdType.MESH)

        def combine(acc_ref, ml_ref, sl):
            m_in = ml_ref[sl, 0]
            l_in = ml_ref[sl, 1]
            m_old = rml[0]
            l_old = rml[1]
            mx = jnp.maximum(m_old, m_in)
            a_old = jnp.exp(m_old - mx)
            a_in = jnp.exp(m_in - mx)
            rml[0] = mx
            rml[1] = l_old * a_old + l_in * a_in
            for bh in range(BH):
                racc[bh * Sq:(bh + 1) * Sq, :] = (
                    racc[bh * Sq:(bh + 1) * Sq, :] * a_old[bh][:, None]
                    + acc_ref[sl, bh * Sq:(bh + 1) * Sq, :].astype(jnp.float32)
                    * a_in[bh][:, None])

        @pl.when(is_even)
        def _even():
            for nbr in (left_e, right_e, my + 1, lt):
                pl.semaphore_signal(bsem, inc=1, device_id=(nbr,),
                                    device_id_type=pl.DeviceIdType.MESH)

            q_blk = lax.broadcasted_iota(jnp.int32, (Sq, SKV_LOC), 0) // 64
            k_blk = my * 2 + lax.broadcasted_iota(jnp.int32, (Sq, SKV_LOC), 1) // 64
            mask = (k_blk % 4) == q_blk
            for b in range(B):
                q_full = jnp.dot(x_ref[b], wq_ref[...],
                                 preferred_element_type=jnp.float32)
                for h in range(Hq):
                    bh = b * Hq + h
                    q_bh = q_full[:, h * Dh:(h + 1) * Dh]
                    k_bh = k_ref[b, :, h, :]
                    s = lax.dot_general(
                        q_bh, k_bh, (((1,), (1,)), ((), ())),
                        preferred_element_type=jnp.float32) * 0.125
                    s = jnp.where(mask, s, NEG)
                    m = jnp.max(s, axis=1)
                    e = jnp.where(mask, jnp.exp(s - m[:, None]), 0.0)
                    lsum = jnp.sum(e, axis=1)
                    a = jnp.dot(e, v_ref[b, :, h, :],
                                preferred_element_type=jnp.float32)
                    for mlr in (cRm, cLm):
                        mlr[0, 0, bh, :] = m
                        mlr[0, 1, bh, :] = lsum
                    rml[0, bh, :] = m
                    rml[1, bh, :] = lsum
                    a_bf = a.astype(jnp.bfloat16)
                    for accr in (cRa, cLa):
                        accr[0, bh * Sq:(bh + 1) * Sq, :] = a_bf
                    racc[bh * Sq:(bh + 1) * Sq, :] = a

            pl.semaphore_wait(bsem, 4)

            def mk1(buf, s_s, s_r, h, dev):
                r = pltpu.make_async_remote_copy(
                    src_ref=buf.at[h], dst_ref=buf.at[h + 1],
                    send_sem=s_s.at[h], recv_sem=s_r.at[h],
                    device_id=(dev,), device_id_type=pl.DeviceIdType.MESH)
                r.start()
                return r

            def mk(acc_ref, ml_ref, a_s, a_r, m_s, m_r, h, dev):
                return (mk1(acc_ref, a_s, a_r, h, dev),
                        mk1(ml_ref, m_s, m_r, h, dev))

            rs = [mk(cRa, cRm, aRs, aRr, mRs, mRr, 0, right_e)]
            ls = [mk(cLa, cLm, aLs, aLr, mLs, mLr, 0, left_e)]
            ga.start()
            gm.start()
            for h in range(R_HOPS - 1):
                rs[h][0].wait_recv()
                if h + 1 < R_HOPS:
                    next_ra = mk1(cRa, aRs, aRr, h + 1, right_e)
                if h < L_HOPS:
                    ls[h][0].wait_recv()
                    if h + 1 < L_HOPS:
                        next_la = mk1(cLa, aLs, aLr, h + 1, left_e)
                rs[h][1].wait_recv()
                if h + 1 < R_HOPS:
                    rs.append((next_ra, mk1(cRm, mRs, mRr, h + 1, right_e)))
                if h < L_HOPS:
                    ls[h][1].wait_recv()
                    if h + 1 < L_HOPS:
                        ls.append((next_la, mk1(cLm, mLs, mLr, h + 1, left_e)))
                combine(cRa, cRm, h + 1)
                if h < L_HOPS:
                    combine(cLa, cLm, h + 1)

            sacc[...] = racc[...]
            sml[...] = rml[...]
            fa.start()
            fm.start()

            rs[R_HOPS - 1][0].wait_recv()
            rs[R_HOPS - 1][1].wait_recv()
            combine(cRa, cRm, R_HOPS)

            for ra, rm in rs + ls:
                ra.wait_send()
                rm.wait_send()
            ga.wait_send()
            gm.wait_send()

        @pl.when(jnp.logical_not(is_even))
        def _odd():
            for nbr in (partner, src_e):
                pl.semaphore_signal(bsem, inc=1, device_id=(nbr,),
                                    device_id_type=pl.DeviceIdType.MESH)
            pl.semaphore_wait(bsem, 2)
            fa.wait_recv()
            fm.wait_recv()
            ga.wait_recv()
            gm.wait_recv()
            combine(cRa, cRm, R_HOPS)

        for b in range(B):
            ob = jnp.zeros((Sq, D_MODEL), jnp.float32)
            for h in range(Hq):
                bh = b * Hq + h
                lsum = rml[1, bh, :]
                ctx = racc[bh * Sq:(bh + 1) * Sq, :] / lsum[:, None]
                ob = ob + jnp.dot(ctx, wo_ref[h * Dh:(h + 1) * Dh, :],
                                  preferred_element_type=jnp.float32)
            out_ref[b] = ob

        @pl.when(is_even)
        def _even_drain():
            fa.wait_send()
            fm.wait_send()

        @functools.partial(pl.run_scoped, ack=pltpu.SemaphoreType.REGULAR)
        def _(ack):
            @pl.when(jnp.logical_not(is_even))
            def _():
                pl.semaphore_signal(ack, inc=1, device_id=(partner,),
                                    device_id_type=pl.DeviceIdType.MESH)

            @pl.when(is_even)
            def _():
                pl.semaphore_wait(ack, 1)

    return pl.pallas_call(
        body,
        out_shape=jax.ShapeDtypeStruct((B, Sq, D_MODEL), jnp.float32),
        in_specs=[pl.BlockSpec(memory_space=pltpu.VMEM)] * 5,
        out_specs=pl.BlockSpec(memory_space=pltpu.VMEM),
        scratch_shapes=[
            pltpu.VMEM((R_HOPS + 1, ROWS, Dh), jnp.bfloat16),
            pltpu.VMEM((R_HOPS + 1, 2, BH, Sq), jnp.float32),
            pltpu.VMEM((L_HOPS + 1, ROWS, Dh), jnp.bfloat16),
            pltpu.VMEM((L_HOPS + 1, 2, BH, Sq), jnp.float32),
            pltpu.SemaphoreType.DMA((R_HOPS,)),
            pltpu.SemaphoreType.DMA((R_HOPS,)),
            pltpu.SemaphoreType.DMA((R_HOPS,)),
            pltpu.SemaphoreType.DMA((R_HOPS,)),
            pltpu.SemaphoreType.DMA((L_HOPS,)),
            pltpu.SemaphoreType.DMA((L_HOPS,)),
            pltpu.SemaphoreType.DMA((L_HOPS,)),
            pltpu.SemaphoreType.DMA((L_HOPS,)),
            pltpu.SemaphoreType.DMA,
            pltpu.SemaphoreType.DMA,
            pltpu.SemaphoreType.DMA,
            pltpu.SemaphoreType.DMA,
            pltpu.SemaphoreType.DMA,
            pltpu.SemaphoreType.DMA,
            pltpu.SemaphoreType.DMA,
            pltpu.SemaphoreType.DMA,
            pltpu.VMEM((ROWS, Dh), jnp.float32),
            pltpu.VMEM((2, BH, Sq), jnp.float32),
            pltpu.VMEM((ROWS, Dh), jnp.float32),
            pltpu.VMEM((2, BH, Sq), jnp.float32),
        ],
        compiler_params=pltpu.CompilerParams(collective_id=0),
    )(x, Wq, K_ext, V_ext, Wo)


# device time: 34305 ns/iter; 1.0674x vs baseline; 1.0674x over previous
import functools

import jax
import jax.numpy as jnp
from jax import lax
from jax.experimental import pallas as pl
from jax.experimental.pallas import tpu as pltpu

N_DEV = 16
B, Sq, Hq, Dh = 2, 128, 4, 64
SKV_LOC = 128
BH = B * Hq
ROWS = BH * Sq
D_MODEL = 512
NEG = -1e9
R_HOPS = 4
L_HOPS = 3


def kernel(x, Wq, K_ext, V_ext, Wo):
    def body(x_ref, wq_ref, k_ref, v_ref, wo_ref, out_ref,
             cRa, cRm, cLa, cLm,
             aRs, aRr, mRs, mRr, aLs, aLr, mLs, mLr,
             fas, far, fms, fmr,
             racc, rml, fin_acc, fin_ml):
        my = lax.axis_index("i")
        is_even = (my % 2) == 0
        partner = jnp.where(is_even, my + 1, my - 1)

        k_idx = jnp.where(my % 4 == 0, my // 4, 4 + (14 - my) // 4)

        def cyc(kk):
            return jnp.where(kk <= 3, 4 * kk, 30 - 4 * kk)

        right_e = cyc((k_idx + 1) % 8)
        left_e = cyc((k_idx + 7) % 8)

        bsem = pltpu.get_barrier_semaphore()

        fa = pltpu.make_async_remote_copy(
            src_ref=fin_acc, dst_ref=fin_acc, send_sem=fas, recv_sem=far,
            device_id=(partner,), device_id_type=pl.DeviceIdType.MESH)
        fm = pltpu.make_async_remote_copy(
            src_ref=fin_ml, dst_ref=fin_ml, send_sem=fms, recv_sem=fmr,
            device_id=(partner,), device_id_type=pl.DeviceIdType.MESH)

        def combine(acc_ref, ml_ref, sl):
            m_in = ml_ref[sl, 0]
            l_in = ml_ref[sl, 1]
            m_old = rml[0]
            l_old = rml[1]
            mx = jnp.maximum(m_old, m_in)
            a_old = jnp.exp(m_old - mx)
            a_in = jnp.exp(m_in - mx)
            rml[0] = mx
            rml[1] = l_old * a_old + l_in * a_in
            for bh in range(BH):
                racc[bh * Sq:(bh + 1) * Sq, :] = (
                    racc[bh * Sq:(bh + 1) * Sq, :] * a_old[bh][:, None]
                    + acc_ref[sl, bh * Sq:(bh + 1) * Sq, :].astype(jnp.float32)
                    * a_in[bh][:, None])

        @pl.when(is_even)
        def _even():
            for nbr in (left_e, right_e, my + 1):
                pl.semaphore_signal(bsem, inc=1, device_id=(nbr,),
                                    device_id_type=pl.DeviceIdType.MESH)

            q_blk = lax.broadcasted_iota(jnp.int32, (Sq, SKV_LOC), 0) // 64
            k_blk = my * 2 + lax.broadcasted_iota(jnp.int32, (Sq, SKV_LOC), 1) // 64
            mask = (k_blk % 4) == q_blk
            for b in range(B):
                q_full = jnp.dot(x_ref[b], wq_ref[...],
                                 preferred_element_type=jnp.float32)
                for h in range(Hq):
                    bh = b * Hq + h
                    q_bh = q_full[:, h * Dh:(h + 1) * Dh]
                    k_bh = k_ref[b, :, h, :]
                    s = lax.dot_general(
                        q_bh, k_bh, (((1,), (1,)), ((), ())),
                        preferred_element_type=jnp.float32) * 0.125
                    s = jnp.where(mask, s, NEG)
                    m = jnp.max(s, axis=1)
                    e = jnp.where(mask, jnp.exp(s - m[:, None]), 0.0)
                    lsum = jnp.sum(e, axis=1)
                    a = jnp.dot(e, v_ref[b, :, h, :],
                                preferred_element_type=jnp.float32)
                    for mlr in (cRm, cLm):
                        mlr[0, 0, bh, :] = m
                        mlr[0, 1, bh, :] = lsum
                    rml[0, bh, :] = m
                    rml[1, bh, :] = lsum
                    a_bf = a.astype(jnp.bfloat16)
                    for accr in (cRa, cLa):
                        accr[0, bh * Sq:(bh + 1) * Sq, :] = a_bf
                    racc[bh * Sq:(bh + 1) * Sq, :] = a

            pl.semaphore_wait(bsem, 2)

            def mk1(buf, s_s, s_r, h, dev):
                r = pltpu.make_async_remote_copy(
                    src_ref=buf.at[h], dst_ref=buf.at[h + 1],
                    send_sem=s_s.at[h], recv_sem=s_r.at[h],
                    device_id=(dev,), device_id_type=pl.DeviceIdType.MESH)
                r.start()
                return r

            def mk(acc_ref, ml_ref, a_s, a_r, m_s, m_r, h, dev):
                return (mk1(acc_ref, a_s, a_r, h, dev),
                        mk1(ml_ref, m_s, m_r, h, dev))

            rs = [mk(cRa, cRm, aRs, aRr, mRs, mRr, 0, right_e)]
            ls = [mk(cLa, cLm, aLs, aLr, mLs, mLr, 0, left_e)]
            for h in range(R_HOPS):
                rs[h][0].wait_recv()
                if h + 1 < R_HOPS:
                    next_ra = mk1(cRa, aRs, aRr, h + 1, right_e)
                if h < L_HOPS:
                    ls[h][0].wait_recv()
                    if h + 1 < L_HOPS:
                        next_la = mk1(cLa, aLs, aLr, h + 1, left_e)
                rs[h][1].wait_recv()
                if h + 1 < R_HOPS:
                    rs.append((next_ra, mk1(cRm, mRs, mRr, h + 1, right_e)))
                if h < L_HOPS:
                    ls[h][1].wait_recv()
                    if h + 1 < L_HOPS:
                        ls.append((next_la, mk1(cLm, mLs, mLr, h + 1, left_e)))
                combine(cRa, cRm, h + 1)
                if h < L_HOPS:
                    combine(cLa, cLm, h + 1)

            pl.semaphore_wait(bsem, 1)
            for bh in range(BH):
                fin_acc[bh * Sq:(bh + 1) * Sq, :] = (
                    racc[bh * Sq:(bh + 1) * Sq, :].astype(jnp.bfloat16))
            fin_ml[...] = rml[...]
            fa.start()
            fm.start()

            for ra, rm in rs + ls:
                ra.wait_send()
                rm.wait_send()

        @pl.when(jnp.logical_not(is_even))
        def _odd():
            pl.semaphore_signal(bsem, inc=1, device_id=(partner,),
                                device_id_type=pl.DeviceIdType.MESH)
            pl.semaphore_wait(bsem, 1)
            fa.wait_recv()
            fm.wait_recv()

        for b in range(B):
            ob = jnp.zeros((Sq, D_MODEL), jnp.float32)
            for h in range(Hq):
                bh = b * Hq + h
                lsum = fin_ml[1, bh, :]
                ctx = (fin_acc[bh * Sq:(bh + 1) * Sq, :].astype(jnp.float32)
                       / lsum[:, None])
                ob = ob + jnp.dot(ctx, wo_ref[h * Dh:(h + 1) * Dh, :],
                                  preferred_element_type=jnp.float32)
            out_ref[b] = ob

        @pl.when(is_even)
        def _even_drain():
            fa.wait_send()
            fm.wait_send()

        @functools.partial(pl.run_scoped, ack=pltpu.SemaphoreType.REGULAR)
        def _(ack):
            @pl.when(jnp.logical_not(is_even))
            def _():
                pl.semaphore_signal(ack, inc=1, device_id=(partner,),
                                    device_id_type=pl.DeviceIdType.MESH)

            @pl.when(is_even)
            def _():
                pl.semaphore_wait(ack, 1)

    return pl.pallas_call(
        body,
        out_shape=jax.ShapeDtypeStruct((B, Sq, D_MODEL), jnp.float32),
        in_specs=[pl.BlockSpec(memory_space=pltpu.VMEM)] * 5,
        out_specs=pl.BlockSpec(memory_space=pltpu.VMEM),
        scratch_shapes=[
            pltpu.VMEM((R_HOPS + 1, ROWS, Dh), jnp.bfloat16),
            pltpu.VMEM((R_HOPS + 1, 2, BH, Sq), jnp.float32),
            pltpu.VMEM((L_HOPS + 1, ROWS, Dh), jnp.bfloat16),
            pltpu.VMEM((L_HOPS + 1, 2, BH, Sq), jnp.float32),
            pltpu.SemaphoreType.DMA((R_HOPS,)),
            pltpu.SemaphoreType.DMA((R_HOPS,)),
            pltpu.SemaphoreType.DMA((R_HOPS,)),
            pltpu.SemaphoreType.DMA((R_HOPS,)),
            pltpu.SemaphoreType.DMA((L_HOPS,)),
            pltpu.SemaphoreType.DMA((L_HOPS,)),
            pltpu.SemaphoreType.DMA((L_HOPS,)),
            pltpu.SemaphoreType.DMA((L_HOPS,)),
            pltpu.SemaphoreType.DMA,
            pltpu.SemaphoreType.DMA,
            pltpu.SemaphoreType.DMA,
            pltpu.SemaphoreType.DMA,
            pltpu.VMEM((ROWS, Dh), jnp.float32),
            pltpu.VMEM((2, BH, Sq), jnp.float32),
            pltpu.VMEM((ROWS, Dh), jnp.bfloat16),
            pltpu.VMEM((2, BH, Sq), jnp.float32),
        ],
        compiler_params=pltpu.CompilerParams(collective_id=0),
    )(x, Wq, K_ext, V_ext, Wo)


# device time: 29551 ns/iter; 1.2391x vs baseline; 1.1609x over previous
import jax
import jax.numpy as jnp
from jax import lax
from jax.experimental import pallas as pl
from jax.experimental.pallas import tpu as pltpu

N_DEV = 16
B, Sq, Hq, Dh = 2, 128, 4, 64
SKV_LOC = 128
BH = B * Hq
HBH = BH // 2
HROWS = HBH * Sq
D_MODEL = 512
NEG = -1e9
R_HOPS = 4
L_HOPS = 3


def kernel(x, Wq, K_ext, V_ext, Wo):
    def body(x_ref, wq_ref, k_ref, v_ref, wo_ref, out_ref,
             cRa, cRm, cLa, cLm,
             aRs, aRr, mRs, mRr, aLs, aLr, mLs, mLr,
             sas, sar, sms, smr, xas, xar, xms, xmr,
             racc, rml, ssa, ssm, sxa, xa, xml, fin_acc, fin_ml):
        my = lax.axis_index("i")
        par = my % 2
        is_even = par == 0
        partner = jnp.where(is_even, my + 1, my - 1)

        k_idx = jnp.where(my % 4 <= 1, my // 4, 4 + (14 + par - my) // 4)

        def cyc(kk):
            return jnp.where(kk <= 3, 4 * kk, 30 - 4 * kk) + par

        right_c = cyc((k_idx + 1) % 8)
        left_c = cyc((k_idx + 7) % 8)

        bsem = pltpu.get_barrier_semaphore()

        seed_a = pltpu.make_async_remote_copy(
            src_ref=ssa, dst_ref=cRa.at[0], send_sem=sas, recv_sem=sar,
            device_id=(partner,), device_id_type=pl.DeviceIdType.MESH)
        seed_m = pltpu.make_async_remote_copy(
            src_ref=ssm, dst_ref=cRm.at[0], send_sem=sms, recv_sem=smr,
            device_id=(partner,), device_id_type=pl.DeviceIdType.MESH)
        ex_a = pltpu.make_async_remote_copy(
            src_ref=sxa, dst_ref=xa, send_sem=xas, recv_sem=xar,
            device_id=(partner,), device_id_type=pl.DeviceIdType.MESH)
        ex_m = pltpu.make_async_remote_copy(
            src_ref=rml, dst_ref=xml, send_sem=xms, recv_sem=xmr,
            device_id=(partner,), device_id_type=pl.DeviceIdType.MESH)

        def combine(acc_ref, ml_ref, sl):
            m_in = ml_ref[sl, 0]
            l_in = ml_ref[sl, 1]
            m_old = rml[0]
            l_old = rml[1]
            mx = jnp.maximum(m_old, m_in)
            a_old = jnp.exp(m_old - mx)
            a_in = jnp.exp(m_in - mx)
            rml[0] = mx
            rml[1] = l_old * a_old + l_in * a_in
            for r in range(HBH):
                racc[r * Sq:(r + 1) * Sq, :] = (
                    racc[r * Sq:(r + 1) * Sq, :] * a_old[r][:, None]
                    + acc_ref[sl, r * Sq:(r + 1) * Sq, :].astype(jnp.float32)
                    * a_in[r][:, None])

        for nbr in (left_c, right_c, partner):
            pl.semaphore_signal(bsem, inc=1, device_id=(nbr,),
                                device_id_type=pl.DeviceIdType.MESH)

        @pl.when(is_even)
        def _even():
            q_blk = lax.broadcasted_iota(jnp.int32, (Sq, SKV_LOC), 0) // 64
            k_blk = my * 2 + lax.broadcasted_iota(jnp.int32, (Sq, SKV_LOC), 1) // 64
            mask = (k_blk % 4) == q_blk
            for b in range(B):
                q_full = jnp.dot(x_ref[b], wq_ref[...],
                                 preferred_element_type=jnp.float32)
                for h in range(Hq):
                    bh = b * Hq + h
                    q_bh = q_full[:, h * Dh:(h + 1) * Dh]
                    k_bh = k_ref[b, :, h, :]
                    s = lax.dot_general(
                        q_bh, k_bh, (((1,), (1,)), ((), ())),
                        preferred_element_type=jnp.float32) * 0.125
                    s = jnp.where(mask, s, NEG)
                    m = jnp.max(s, axis=1)
                    e = jnp.where(mask, jnp.exp(s - m[:, None]), 0.0)
                    lsum = jnp.sum(e, axis=1)
                    a = jnp.dot(e, v_ref[b, :, h, :],
                                preferred_element_type=jnp.float32)
                    a_bf = a.astype(jnp.bfloat16)
                    if bh < HBH:
                        for mlr in (cRm, cLm):
                            mlr[0, 0, bh, :] = m
                            mlr[0, 1, bh, :] = lsum
                        rml[0, bh, :] = m
                        rml[1, bh, :] = lsum
                        for accr in (cRa, cLa):
                            accr[0, bh * Sq:(bh + 1) * Sq, :] = a_bf
                        racc[bh * Sq:(bh + 1) * Sq, :] = a
                    else:
                        r = bh - HBH
                        ssm[0, r, :] = m
                        ssm[1, r, :] = lsum
                        ssa[r * Sq:(r + 1) * Sq, :] = a_bf
            pl.semaphore_wait(bsem, 3)
            seed_a.start()
            seed_m.start()

        @pl.when(jnp.logical_not(is_even))
        def _odd():
            pl.semaphore_wait(bsem, 3)
            seed_a.wait_recv()
            seed_m.wait_recv()
            cLa[0, :, :] = cRa[0, :, :]
            cLm[0, :, :, :] = cRm[0, :, :, :]
            racc[...] = cRa[0].astype(jnp.float32)
            rml[...] = cRm[0, :, :, :]

        def mk1(buf, s_s, s_r, h, dev):
            r = pltpu.make_async_remote_copy(
                src_ref=buf.at[h], dst_ref=buf.at[h + 1],
                send_sem=s_s.at[h], recv_sem=s_r.at[h],
                device_id=(dev,), device_id_type=pl.DeviceIdType.MESH)
            r.start()
            return r

        def mk(acc_ref, ml_ref, a_s, a_r, m_s, m_r, h, dev):
            return (mk1(acc_ref, a_s, a_r, h, dev),
                    mk1(ml_ref, m_s, m_r, h, dev))

        rs = [mk(cRa, cRm, aRs, aRr, mRs, mRr, 0, right_c)]
        ls = [mk(cLa, cLm, aLs, aLr, mLs, mLr, 0, left_c)]
        for h in range(R_HOPS):
            rs[h][0].wait_recv()
            if h + 1 < R_HOPS:
                next_ra = mk1(cRa, aRs, aRr, h + 1, right_c)
            if h < L_HOPS:
                ls[h][0].wait_recv()
                if h + 1 < L_HOPS:
                    next_la = mk1(cLa, aLs, aLr, h + 1, left_c)
            rs[h][1].wait_recv()
            if h + 1 < R_HOPS:
                rs.append((next_ra, mk1(cRm, mRs, mRr, h + 1, right_c)))
            if h < L_HOPS:
                ls[h][1].wait_recv()
                if h + 1 < L_HOPS:
                    ls.append((next_la, mk1(cLm, mLs, mLr, h + 1, left_c)))
            combine(cRa, cRm, h + 1)
            if h < L_HOPS:
                combine(cLa, cLm, h + 1)

        for ra, rm in rs + ls:
            ra.wait_send()
            rm.wait_send()

        @pl.when(is_even)
        def _even_seed_drain():
            seed_a.wait_send()
            seed_m.wait_send()

        sxa[...] = racc[...].astype(jnp.bfloat16)
        ex_a.start()
        ex_m.start()
        ex_a.wait_recv()
        ex_m.wait_recv()

        @pl.when(is_even)
        def _asm_even():
            fin_acc[0:HROWS, :] = racc[...]
            fin_acc[HROWS:, :] = xa[...].astype(jnp.float32)
            fin_ml[:, 0:HBH, :] = rml[...]
            fin_ml[:, HBH:, :] = xml[...]

        @pl.when(jnp.logical_not(is_even))
        def _asm_odd():
            fin_acc[0:HROWS, :] = xa[...].astype(jnp.float32)
            fin_acc[HROWS:, :] = racc[...]
            fin_ml[:, 0:HBH, :] = xml[...]
            fin_ml[:, HBH:, :] = rml[...]

        for b in range(B):
            ob = jnp.zeros((Sq, D_MODEL), jnp.float32)
            for h in range(Hq):
                bh = b * Hq + h
                lsum = fin_ml[1, bh, :]
                ctx = fin_acc[bh * Sq:(bh + 1) * Sq, :] / lsum[:, None]
                ob = ob + jnp.dot(ctx, wo_ref[h * Dh:(h + 1) * Dh, :],
                                  preferred_element_type=jnp.float32)
            out_ref[b] = ob

        ex_a.wait_send()
        ex_m.wait_send()

    return pl.pallas_call(
        body,
        out_shape=jax.ShapeDtypeStruct((B, Sq, D_MODEL), jnp.float32),
        in_specs=[pl.BlockSpec(memory_space=pltpu.VMEM)] * 5,
        out_specs=pl.BlockSpec(memory_space=pltpu.VMEM),
        scratch_shapes=[
            pltpu.VMEM((R_HOPS + 1, HROWS, Dh), jnp.bfloat16),
            pltpu.VMEM((R_HOPS + 1, 2, HBH, Sq), jnp.float32),
            pltpu.VMEM((L_HOPS + 1, HROWS, Dh), jnp.bfloat16),
            pltpu.VMEM((L_HOPS + 1, 2, HBH, Sq), jnp.float32),
            pltpu.SemaphoreType.DMA((R_HOPS,)),
            pltpu.SemaphoreType.DMA((R_HOPS,)),
            pltpu.SemaphoreType.DMA((R_HOPS,)),
            pltpu.SemaphoreType.DMA((R_HOPS,)),
            pltpu.SemaphoreType.DMA((L_HOPS,)),
            pltpu.SemaphoreType.DMA((L_HOPS,)),
            pltpu.SemaphoreType.DMA((L_HOPS,)),
            pltpu.SemaphoreType.DMA((L_HOPS,)),
            pltpu.SemaphoreType.DMA,
            pltpu.SemaphoreType.DMA,
            pltpu.SemaphoreType.DMA,
            pltpu.SemaphoreType.DMA,
            pltpu.SemaphoreType.DMA,
            pltpu.SemaphoreType.DMA,
            pltpu.SemaphoreType.DMA,
            pltpu.SemaphoreType.DMA,
            pltpu.VMEM((HROWS, Dh), jnp.float32),
            pltpu.VMEM((2, HBH, Sq), jnp.float32),
            pltpu.VMEM((HROWS, Dh), jnp.bfloat16),
            pltpu.VMEM((2, HBH, Sq), jnp.float32),
            pltpu.VMEM((HROWS, Dh), jnp.bfloat16),
            pltpu.VMEM((HROWS, Dh), jnp.bfloat16),
            pltpu.VMEM((2, HBH, Sq), jnp.float32),
            pltpu.VMEM((BH * Sq, Dh), jnp.float32),
            pltpu.VMEM((2, BH, Sq), jnp.float32),
        ],
        compiler_params=pltpu.CompilerParams(collective_id=0),
    )(x, Wq, K_ext, V_ext, Wo)


# device time: 29360 ns/iter; 1.2471x vs baseline; 1.0065x over previous
import jax
import jax.numpy as jnp
from jax import lax
from jax.experimental import pallas as pl
from jax.experimental.pallas import tpu as pltpu

N_DEV = 16
B, Sq, Hq, Dh = 2, 128, 4, 64
SKV_LOC = 128
BH = B * Hq
HBH = BH // 2
HROWS = HBH * Sq
D_MODEL = 512
NEG = -1e9
R_HOPS = 4
L_HOPS = 3


def kernel(x, Wq, K_ext, V_ext, Wo):
    def body(x_ref, wq_ref, k_ref, v_ref, wo_ref, out_ref,
             cRa, cRm, cLa, cLm,
             aRs, aRr, mRs, mRr, aLs, aLr, mLs, mLr,
             sas, sar, sms, smr, xas, xar, xms, xmr,
             racc, rml, ssa, ssm, sxa, xa, xml):
        my = lax.axis_index("i")
        par = my % 2
        is_even = par == 0
        partner = jnp.where(is_even, my + 1, my - 1)

        k_idx = jnp.where(my % 4 <= 1, my // 4, 4 + (14 + par - my) // 4)

        def cyc(kk):
            return jnp.where(kk <= 3, 4 * kk, 30 - 4 * kk) + par

        right_c = cyc((k_idx + 1) % 8)
        left_c = cyc((k_idx + 7) % 8)

        bsem = pltpu.get_barrier_semaphore()

        seed_a = pltpu.make_async_remote_copy(
            src_ref=ssa, dst_ref=cRa.at[0], send_sem=sas, recv_sem=sar,
            device_id=(partner,), device_id_type=pl.DeviceIdType.MESH)
        seed_m = pltpu.make_async_remote_copy(
            src_ref=ssm, dst_ref=cRm.at[0], send_sem=sms, recv_sem=smr,
            device_id=(partner,), device_id_type=pl.DeviceIdType.MESH)
        ex_a = pltpu.make_async_remote_copy(
            src_ref=sxa, dst_ref=xa, send_sem=xas, recv_sem=xar,
            device_id=(partner,), device_id_type=pl.DeviceIdType.MESH)
        ex_m = pltpu.make_async_remote_copy(
            src_ref=rml, dst_ref=xml, send_sem=xms, recv_sem=xmr,
            device_id=(partner,), device_id_type=pl.DeviceIdType.MESH)

        def combine(acc_ref, ml_ref, sl):
            m_in = ml_ref[sl, 0]
            l_in = ml_ref[sl, 1]
            m_old = rml[0]
            l_old = rml[1]
            mx = jnp.maximum(m_old, m_in)
            a_old = jnp.exp(m_old - mx)
            a_in = jnp.exp(m_in - mx)
            rml[0] = mx
            rml[1] = l_old * a_old + l_in * a_in
            for r in range(HBH):
                racc[r * Sq:(r + 1) * Sq, :] = (
                    racc[r * Sq:(r + 1) * Sq, :] * a_old[r][:, None]
                    + acc_ref[sl, r * Sq:(r + 1) * Sq, :].astype(jnp.float32)
                    * a_in[r][:, None])

        for nbr in (left_c, right_c, partner):
            pl.semaphore_signal(bsem, inc=1, device_id=(nbr,),
                                device_id_type=pl.DeviceIdType.MESH)

        @pl.when(is_even)
        def _even():
            q_blk = lax.broadcasted_iota(jnp.int32, (Sq, SKV_LOC), 0) // 64
            k_blk = my * 2 + lax.broadcasted_iota(jnp.int32, (Sq, SKV_LOC), 1) // 64
            mask = (k_blk % 4) == q_blk
            for b in range(B):
                q_full = jnp.dot(x_ref[b], wq_ref[...],
                                 preferred_element_type=jnp.float32)
                for h in range(Hq):
                    bh = b * Hq + h
                    q_bh = q_full[:, h * Dh:(h + 1) * Dh]
                    k_bh = k_ref[b, :, h, :]
                    s = lax.dot_general(
                        q_bh, k_bh, (((1,), (1,)), ((), ())),
                        preferred_element_type=jnp.float32) * 0.125
                    s = jnp.where(mask, s, NEG)
                    m = jnp.max(s, axis=1)
                    e = jnp.where(mask, jnp.exp(s - m[:, None]), 0.0)
                    lsum = jnp.sum(e, axis=1)
                    a = jnp.dot(e, v_ref[b, :, h, :],
                                preferred_element_type=jnp.float32)
                    a_bf = a.astype(jnp.bfloat16)
                    if bh < HBH:
                        for mlr in (cRm, cLm):
                            mlr[0, 0, bh, :] = m
                            mlr[0, 1, bh, :] = lsum
                        rml[0, bh, :] = m
                        rml[1, bh, :] = lsum
                        for accr in (cRa, cLa):
                            accr[0, bh * Sq:(bh + 1) * Sq, :] = a_bf
                        racc[bh * Sq:(bh + 1) * Sq, :] = a
                    else:
                        r = bh - HBH
                        ssm[0, r, :] = m
                        ssm[1, r, :] = lsum
                        ssa[r * Sq:(r + 1) * Sq, :] = a_bf
            pl.semaphore_wait(bsem, 3)
            seed_a.start()
            seed_m.start()

        @pl.when(jnp.logical_not(is_even))
        def _odd():
            pl.semaphore_wait(bsem, 3)
            seed_a.wait_recv()
            seed_m.wait_recv()
            cLa[0, :, :] = cRa[0, :, :]
            cLm[0, :, :, :] = cRm[0, :, :, :]
            racc[...] = cRa[0].astype(jnp.float32)
            rml[...] = cRm[0, :, :, :]

        def mk1(buf, s_s, s_r, h, dev):
            r = pltpu.make_async_remote_copy(
                src_ref=buf.at[h], dst_ref=buf.at[h + 1],
                send_sem=s_s.at[h], recv_sem=s_r.at[h],
                device_id=(dev,), device_id_type=pl.DeviceIdType.MESH)
            r.start()
            return r

        def mk(acc_ref, ml_ref, a_s, a_r, m_s, m_r, h, dev):
            return (mk1(acc_ref, a_s, a_r, h, dev),
                    mk1(ml_ref, m_s, m_r, h, dev))

        rs = [mk(cRa, cRm, aRs, aRr, mRs, mRr, 0, right_c)]
        ls = [mk(cLa, cLm, aLs, aLr, mLs, mLr, 0, left_c)]
        for h in range(R_HOPS):
            rs[h][0].wait_recv()
            if h + 1 < R_HOPS:
                next_ra = mk1(cRa, aRs, aRr, h + 1, right_c)
            if h < L_HOPS:
                ls[h][0].wait_recv()
                if h + 1 < L_HOPS:
                    next_la = mk1(cLa, aLs, aLr, h + 1, left_c)
            rs[h][1].wait_recv()
            if h + 1 < R_HOPS:
                rs.append((next_ra, mk1(cRm, mRs, mRr, h + 1, right_c)))
            if h < L_HOPS:
                ls[h][1].wait_recv()
                if h + 1 < L_HOPS:
                    ls.append((next_la, mk1(cLm, mLs, mLr, h + 1, left_c)))
            combine(cRa, cRm, h + 1)
            if h < L_HOPS:
                combine(cLa, cLm, h + 1)

        for ra, rm in rs + ls:
            ra.wait_send()
            rm.wait_send()

        @pl.when(is_even)
        def _even_seed_drain():
            seed_a.wait_send()
            seed_m.wait_send()

        sxa[...] = racc[...].astype(jnp.bfloat16)
        ex_a.start()
        ex_m.start()

        def half_out(acc_row, l_row):
            ob = jnp.zeros((Sq, D_MODEL), jnp.float32)
            for h in range(Hq):
                ctx = acc_row(h) / l_row(h)[:, None]
                ob = ob + jnp.dot(ctx, wo_ref[h * Dh:(h + 1) * Dh, :],
                                  preferred_element_type=jnp.float32)
            return ob

        own = half_out(lambda h: racc[h * Sq:(h + 1) * Sq, :],
                       lambda h: rml[1, h, :])

        @pl.when(is_even)
        def _own_even():
            out_ref[0] = own

        @pl.when(jnp.logical_not(is_even))
        def _own_odd():
            out_ref[1] = own

        ex_a.wait_recv()
        ex_m.wait_recv()
        other = half_out(
            lambda h: xa[h * Sq:(h + 1) * Sq, :].astype(jnp.float32),
            lambda h: xml[1, h, :])

        @pl.when(is_even)
        def _other_even():
            out_ref[1] = other

        @pl.when(jnp.logical_not(is_even))
        def _other_odd():
            out_ref[0] = other

        ex_a.wait_send()
        ex_m.wait_send()

    return pl.pallas_call(
        body,
        out_shape=jax.ShapeDtypeStruct((B, Sq, D_MODEL), jnp.float32),
        in_specs=[pl.BlockSpec(memory_space=pltpu.VMEM)] * 5,
        out_specs=pl.BlockSpec(memory_space=pltpu.VMEM),
        scratch_shapes=[
            pltpu.VMEM((R_HOPS + 1, HROWS, Dh), jnp.bfloat16),
            pltpu.VMEM((R_HOPS + 1, 2, HBH, Sq), jnp.float32),
            pltpu.VMEM((L_HOPS + 1, HROWS, Dh), jnp.bfloat16),
            pltpu.VMEM((L_HOPS + 1, 2, HBH, Sq), jnp.float32),
            pltpu.SemaphoreType.DMA((R_HOPS,)),
            pltpu.SemaphoreType.DMA((R_HOPS,)),
            pltpu.SemaphoreType.DMA((R_HOPS,)),
            pltpu.SemaphoreType.DMA((R_HOPS,)),
            pltpu.SemaphoreType.DMA((L_HOPS,)),
            pltpu.SemaphoreType.DMA((L_HOPS,)),
            pltpu.SemaphoreType.DMA((L_HOPS,)),
            pltpu.SemaphoreType.DMA((L_HOPS,)),
            pltpu.SemaphoreType.DMA,
            pltpu.SemaphoreType.DMA,
            pltpu.SemaphoreType.DMA,
            pltpu.SemaphoreType.DMA,
            pltpu.SemaphoreType.DMA,
            pltpu.SemaphoreType.DMA,
            pltpu.SemaphoreType.DMA,
            pltpu.SemaphoreType.DMA,
            pltpu.VMEM((HROWS, Dh), jnp.float32),
            pltpu.VMEM((2, HBH, Sq), jnp.float32),
            pltpu.VMEM((HROWS, Dh), jnp.bfloat16),
            pltpu.VMEM((2, HBH, Sq), jnp.float32),
            pltpu.VMEM((HROWS, Dh), jnp.bfloat16),
            pltpu.VMEM((HROWS, Dh), jnp.bfloat16),
            pltpu.VMEM((2, HBH, Sq), jnp.float32),
        ],
        compiler_params=pltpu.CompilerParams(collective_id=0),
    )(x, Wq, K_ext, V_ext, Wo)


# device time: 28367 ns/iter; 1.2908x vs baseline; 1.0350x over previous
import jax
import jax.numpy as jnp
from jax import lax
from jax.experimental import pallas as pl
from jax.experimental.pallas import tpu as pltpu

N_DEV = 16
B, Sq, Hq, Dh = 2, 128, 4, 64
SKV_LOC = 128
BH = B * Hq
HBH = BH // 2
HROWS = HBH * Sq
D_MODEL = 512
NEG = -1e9
R_HOPS = 4
L_HOPS = 3


def kernel(x, Wq, K_ext, V_ext, Wo):
    def body(x_ref, wq_ref, k_ref, v_ref, wo_ref, out_ref,
             cRa, cRm, cLa, cLm,
             aRs, aRr, mRs, mRr, aLs, aLr, mLs, mLr,
             sas, sar, sms, smr, xas, xar, xms, xmr,
             racc, rml, ssa, ssm, sxa, xa, xml):
        my = lax.axis_index("i")
        par = my % 2
        is_even = par == 0
        partner = jnp.where(is_even, my + 1, my - 1)

        k_idx = jnp.where(my % 4 <= 1, my // 4, 4 + (14 + par - my) // 4)

        def cyc(kk):
            return jnp.where(kk <= 3, 4 * kk, 30 - 4 * kk) + par

        right_c = cyc((k_idx + 1) % 8)
        left_c = cyc((k_idx + 7) % 8)

        bsem = pltpu.get_barrier_semaphore()

        seed_a = pltpu.make_async_remote_copy(
            src_ref=ssa, dst_ref=cRa.at[0], send_sem=sas, recv_sem=sar,
            device_id=(partner,), device_id_type=pl.DeviceIdType.MESH)
        seed_m = pltpu.make_async_remote_copy(
            src_ref=ssm, dst_ref=cRm.at[0], send_sem=sms, recv_sem=smr,
            device_id=(partner,), device_id_type=pl.DeviceIdType.MESH)
        ex_a = pltpu.make_async_remote_copy(
            src_ref=sxa, dst_ref=xa, send_sem=xas, recv_sem=xar,
            device_id=(partner,), device_id_type=pl.DeviceIdType.MESH)
        ex_m = pltpu.make_async_remote_copy(
            src_ref=rml, dst_ref=xml, send_sem=xms, recv_sem=xmr,
            device_id=(partner,), device_id_type=pl.DeviceIdType.MESH)

        def combine(acc_ref, ml_ref, sl):
            m_in = ml_ref[sl, 0]
            l_in = ml_ref[sl, 1]
            m_old = rml[0]
            l_old = rml[1]
            mx = jnp.maximum(m_old, m_in)
            a_old = jnp.exp(m_old - mx)
            a_in = jnp.exp(m_in - mx)
            rml[0] = mx
            rml[1] = l_old * a_old + l_in * a_in
            for r in range(HBH):
                racc[r * Sq:(r + 1) * Sq, :] = (
                    racc[r * Sq:(r + 1) * Sq, :] * a_old[r][:, None]
                    + acc_ref[sl, r * Sq:(r + 1) * Sq, :].astype(jnp.float32)
                    * a_in[r][:, None])

        for nbr in (left_c, right_c, partner):
            pl.semaphore_signal(bsem, inc=1, device_id=(nbr,),
                                device_id_type=pl.DeviceIdType.MESH)

        @pl.when(is_even)
        def _even():
            q_blk = lax.broadcasted_iota(jnp.int32, (Sq, SKV_LOC), 0) // 64
            k_blk = my * 2 + lax.broadcasted_iota(jnp.int32, (Sq, SKV_LOC), 1) // 64
            mask = (k_blk % 4) == q_blk
            for b in (1, 0):
                q_full = jnp.dot(x_ref[b], wq_ref[...],
                                 preferred_element_type=jnp.float32)
                for h in range(Hq):
                    bh = b * Hq + h
                    q_bh = q_full[:, h * Dh:(h + 1) * Dh]
                    k_bh = k_ref[b, :, h, :]
                    s = lax.dot_general(
                        q_bh, k_bh, (((1,), (1,)), ((), ())),
                        preferred_element_type=jnp.float32) * 0.125
                    s = jnp.where(mask, s, NEG)
                    m = jnp.max(s, axis=1)
                    e = jnp.where(mask, jnp.exp(s - m[:, None]), 0.0)
                    lsum = jnp.sum(e, axis=1)
                    a = jnp.dot(e, v_ref[b, :, h, :],
                                preferred_element_type=jnp.float32)
                    a_bf = a.astype(jnp.bfloat16)
                    if bh < HBH:
                        for mlr in (cRm, cLm):
                            mlr[0, 0, bh, :] = m
                            mlr[0, 1, bh, :] = lsum
                        rml[0, bh, :] = m
                        rml[1, bh, :] = lsum
                        for accr in (cRa, cLa):
                            accr[0, bh * Sq:(bh + 1) * Sq, :] = a_bf
                        racc[bh * Sq:(bh + 1) * Sq, :] = a
                    else:
                        r = bh - HBH
                        ssm[0, r, :] = m
                        ssm[1, r, :] = lsum
                        ssa[r * Sq:(r + 1) * Sq, :] = a_bf
                if b == 1:
                    pl.semaphore_wait(bsem, 3)
                    seed_a.start()
                    seed_m.start()

        @pl.when(jnp.logical_not(is_even))
        def _odd():
            pl.semaphore_wait(bsem, 3)
            seed_a.wait_recv()
            seed_m.wait_recv()
            cLa[0, :, :] = cRa[0, :, :]
            cLm[0, :, :, :] = cRm[0, :, :, :]
            racc[...] = cRa[0].astype(jnp.float32)
            rml[...] = cRm[0, :, :, :]

        def mk1(buf, s_s, s_r, h, dev):
            r = pltpu.make_async_remote_copy(
                src_ref=buf.at[h], dst_ref=buf.at[h + 1],
                send_sem=s_s.at[h], recv_sem=s_r.at[h],
                device_id=(dev,), device_id_type=pl.DeviceIdType.MESH)
            r.start()
            return r

        def mk(acc_ref, ml_ref, a_s, a_r, m_s, m_r, h, dev):
            return (mk1(acc_ref, a_s, a_r, h, dev),
                    mk1(ml_ref, m_s, m_r, h, dev))

        rs = [mk(cRa, cRm, aRs, aRr, mRs, mRr, 0, right_c)]
        ls = [mk(cLa, cLm, aLs, aLr, mLs, mLr, 0, left_c)]
        for h in range(R_HOPS):
            rs[h][0].wait_recv()
            if h + 1 < R_HOPS:
                next_ra = mk1(cRa, aRs, aRr, h + 1, right_c)
            if h < L_HOPS:
                ls[h][0].wait_recv()
                if h + 1 < L_HOPS:
                    next_la = mk1(cLa, aLs, aLr, h + 1, left_c)
            rs[h][1].wait_recv()
            if h + 1 < R_HOPS:
                rs.append((next_ra, mk1(cRm, mRs, mRr, h + 1, right_c)))
            if h < L_HOPS:
                ls[h][1].wait_recv()
                if h + 1 < L_HOPS:
                    ls.append((next_la, mk1(cLm, mLs, mLr, h + 1, left_c)))
            combine(cRa, cRm, h + 1)
            if h < L_HOPS:
                combine(cLa, cLm, h + 1)

        for ra, rm in rs + ls:
            ra.wait_send()
            rm.wait_send()

        @pl.when(is_even)
        def _even_seed_drain():
            seed_a.wait_send()
            seed_m.wait_send()

        sxa[...] = racc[...].astype(jnp.bfloat16)
        ex_a.start()
        ex_m.start()

        def half_out(acc_row, l_row):
            ob = jnp.zeros((Sq, D_MODEL), jnp.float32)
            for h in range(Hq):
                ctx = acc_row(h) / l_row(h)[:, None]
                ob = ob + jnp.dot(ctx, wo_ref[h * Dh:(h + 1) * Dh, :],
                                  preferred_element_type=jnp.float32)
            return ob

        own = half_out(lambda h: racc[h * Sq:(h + 1) * Sq, :],
                       lambda h: rml[1, h, :])

        @pl.when(is_even)
        def _own_even():
            out_ref[0] = own

        @pl.when(jnp.logical_not(is_even))
        def _own_odd():
            out_ref[1] = own

        ex_a.wait_recv()
        ex_m.wait_recv()
        other = half_out(
            lambda h: xa[h * Sq:(h + 1) * Sq, :].astype(jnp.float32),
            lambda h: xml[1, h, :])

        @pl.when(is_even)
        def _other_even():
            out_ref[1] = other

        @pl.when(jnp.logical_not(is_even))
        def _other_odd():
            out_ref[0] = other

        ex_a.wait_send()
        ex_m.wait_send()

    return pl.pallas_call(
        body,
        out_shape=jax.ShapeDtypeStruct((B, Sq, D_MODEL), jnp.float32),
        in_specs=[pl.BlockSpec(memory_space=pltpu.VMEM)] * 5,
        out_specs=pl.BlockSpec(memory_space=pltpu.VMEM),
        scratch_shapes=[
            pltpu.VMEM((R_HOPS + 1, HROWS, Dh), jnp.bfloat16),
            pltpu.VMEM((R_HOPS + 1, 2, HBH, Sq), jnp.float32),
            pltpu.VMEM((L_HOPS + 1, HROWS, Dh), jnp.bfloat16),
            pltpu.VMEM((L_HOPS + 1, 2, HBH, Sq), jnp.float32),
            pltpu.SemaphoreType.DMA((R_HOPS,)),
            pltpu.SemaphoreType.DMA((R_HOPS,)),
            pltpu.SemaphoreType.DMA((R_HOPS,)),
            pltpu.SemaphoreType.DMA((R_HOPS,)),
            pltpu.SemaphoreType.DMA((L_HOPS,)),
            pltpu.SemaphoreType.DMA((L_HOPS,)),
            pltpu.SemaphoreType.DMA((L_HOPS,)),
            pltpu.SemaphoreType.DMA((L_HOPS,)),
            pltpu.SemaphoreType.DMA,
            pltpu.SemaphoreType.DMA,
            pltpu.SemaphoreType.DMA,
            pltpu.SemaphoreType.DMA,
            pltpu.SemaphoreType.DMA,
            pltpu.SemaphoreType.DMA,
            pltpu.SemaphoreType.DMA,
            pltpu.SemaphoreType.DMA,
            pltpu.VMEM((HROWS, Dh), jnp.float32),
            pltpu.VMEM((2, HBH, Sq), jnp.float32),
            pltpu.VMEM((HROWS, Dh), jnp.bfloat16),
            pltpu.VMEM((2, HBH, Sq), jnp.float32),
            pltpu.VMEM((HROWS, Dh), jnp.bfloat16),
            pltpu.VMEM((HROWS, Dh), jnp.bfloat16),
            pltpu.VMEM((2, HBH, Sq), jnp.float32),
        ],
        compiler_params=pltpu.CompilerParams(collective_id=0),
    )(x, Wq, K_ext, V_ext, Wo)
